# Initial kernel scaffold; baseline (speedup 1.0000x reference)
import jax
import jax.numpy as jnp
from jax.experimental import pallas as pl


def kernel(query, Wv, bv, Woff, boff, Wattn, battn, Wout, bout):
    raise NotImplementedError("write your pallas kernel here")


# speedup vs baseline: 1276.3065x; 1276.3065x over previous
import functools

import numpy as np
import jax
import jax.numpy as jnp
from jax import lax
from jax.experimental import pallas as pl
from jax.experimental.pallas import tpu as pltpu
from jax.experimental.pallas import tpu_sc as plsc

BEV = 200
NQ = BEV * BEV
E = 128
NH = 8
HD = 16
NPAIR = NQ * NH

BLK = 1000
GRID = NQ // BLK

NC, NS = 2, 16
NW = NC * NS
PW = NPAIR // NW
KP = 200
NCH = PW // KP
NJ = KP * 16 // 128
TILE_IDXROWS = PW * 16 // 128

_HIGH = jax.lax.Precision.HIGHEST

_G4 = np.kron(np.eye(NH, dtype=np.float32), np.ones((4, 4), np.float32))
_P = np.zeros((4, 32, 128), np.float32)
for _c in range(4):
    for _h in range(NH):
        for _p in range(4):
            _P[_c, _h * 4 + _p, _h * 16 + _c * 4 + _p] = 1.0


def _dot(a, b):
    return jax.lax.dot_general(a, b, (((1,), (0,)), ((), ())),
                               precision=_HIGH,
                               preferred_element_type=jnp.float32)


def _pre_body(q_ref, wv_ref, bv_ref, wox_ref, box_ref, woy_ref, boy_ref,
              wat_ref, bat_ref, g4_ref, p0_ref, p1_ref, p2_ref, p3_ref,
              val_ref, idx_ref, wts_ref):
    i = pl.program_id(0)
    q = q_ref[...]
    val_ref[...] = _dot(q, wv_ref[...]) + bv_ref[...]

    offx = _dot(q, wox_ref[...]) + box_ref[...]
    offy = _dot(q, woy_ref[...]) + boy_ref[...]
    lg = _dot(q, wat_ref[...]) + bat_ref[...]
    m = jnp.max(lg, axis=1, keepdims=True)
    e = jnp.exp(lg - m)
    aw = e / _dot(e, g4_ref[...])

    qi = i * BLK + lax.broadcasted_iota(jnp.int32, (BLK, 32), 0)
    refx = (qi % BEV).astype(jnp.float32) * (1.0 / 199.0)
    refy = (qi // BEV).astype(jnp.float32) * (1.0 / 199.0)
    x = refx * 200.0 - 0.5 + offx
    y = refy * 200.0 - 0.5 + offy

    x0 = jnp.floor(x)
    y0 = jnp.floor(y)
    fx = x - x0
    fy = y - y0
    vx0 = ((x0 >= 0.0) & (x0 <= 199.0)).astype(jnp.float32)
    vx1 = ((x0 >= -1.0) & (x0 <= 198.0)).astype(jnp.float32)
    vy0 = ((y0 >= 0.0) & (y0 <= 199.0)).astype(jnp.float32)
    vy1 = ((y0 >= -1.0) & (y0 <= 198.0)).astype(jnp.float32)
    xc0 = jnp.clip(x0, 0.0, 199.0)
    xc1 = jnp.clip(x0 + 1.0, 0.0, 199.0)
    yc0 = jnp.clip(y0, 0.0, 199.0)
    yc1 = jnp.clip(y0 + 1.0, 0.0, 199.0)

    hl = (lax.broadcasted_iota(jnp.int32, (BLK, 32), 1) // 4).astype(jnp.float32)
    b00 = (yc0 * 200.0 + xc0) * 8.0 + hl
    b10 = (yc0 * 200.0 + xc1) * 8.0 + hl
    b01 = (yc1 * 200.0 + xc0) * 8.0 + hl
    b11 = (yc1 * 200.0 + xc1) * 8.0 + hl
    w00 = aw * ((1.0 - fx) * (1.0 - fy)) * (vx0 * vy0)
    w10 = aw * (fx * (1.0 - fy)) * (vx1 * vy0)
    w01 = aw * ((1.0 - fx) * fy) * (vx0 * vy1)
    w11 = aw * (fx * fy) * (vx1 * vy1)

    p0, p1, p2, p3 = p0_ref[...], p1_ref[...], p2_ref[...], p3_ref[...]
    idxf = _dot(b00, p0) + _dot(b10, p1) + _dot(b01, p2) + _dot(b11, p3)
    idx_ref[...] = (idxf + 0.5).astype(jnp.int32)
    wts_ref[...] = _dot(w00, p0) + _dot(w10, p1) + _dot(w01, p2) + _dot(w11, p3)


def _post_body(s_ref, wo_ref, bo_ref, q_ref, out_ref):
    out_ref[...] = (_dot(s_ref[...], wo_ref[...]) + bo_ref[...]
                    + 2.0 * q_ref[...])


def _full(shape):
    return pl.BlockSpec(shape, lambda i: (0, 0))


_pre_call = pl.pallas_call(
    _pre_body,
    grid=(GRID,),
    in_specs=[
        pl.BlockSpec((BLK, E), lambda i: (i, 0)),
        _full((E, E)), _full((1, E)),
        _full((E, 32)), _full((1, 32)),
        _full((E, 32)), _full((1, 32)),
        _full((E, 32)), _full((1, 32)),
        _full((32, 32)),
        _full((32, 128)), _full((32, 128)), _full((32, 128)), _full((32, 128)),
    ],
    out_specs=[
        pl.BlockSpec((BLK, E), lambda i: (i, 0)),
        pl.BlockSpec((BLK, E), lambda i: (i, 0)),
        pl.BlockSpec((BLK, E), lambda i: (i, 0)),
    ],
    out_shape=[
        jax.ShapeDtypeStruct((NQ, E), jnp.float32),
        jax.ShapeDtypeStruct((NQ, E), jnp.int32),
        jax.ShapeDtypeStruct((NQ, E), jnp.float32),
    ],
)

_post_call = pl.pallas_call(
    _post_body,
    grid=(GRID,),
    in_specs=[
        pl.BlockSpec((BLK, E), lambda i: (i, 0)),
        _full((E, E)), _full((1, E)),
        pl.BlockSpec((BLK, E), lambda i: (i, 0)),
    ],
    out_specs=pl.BlockSpec((BLK, E), lambda i: (i, 0)),
    out_shape=jax.ShapeDtypeStruct((NQ, E), jnp.float32),
)


def _sc_body(tab_ref, idx_ref, w_ref, out_ref, idx_v, w_v, rows_v, out_v, sem):
    wid = lax.axis_index("s") * NC + lax.axis_index("c")

    def chunk(ch, carry):
        rowb = wid * TILE_IDXROWS + ch * NJ
        pltpu.sync_copy(idx_ref.at[pl.ds(rowb, NJ)], idx_v)
        pltpu.sync_copy(w_ref.at[pl.ds(rowb, NJ)], w_v)
        cps = [pltpu.async_copy(tab_ref.at[idx_v.at[j]],
                                rows_v.at[pl.ds(j * 128, 128)], sem)
               for j in range(NJ)]
        for cp in cps:
            cp.wait()

        def pair(i, c2):
            rr = i // 8
            cb = (i % 8) * 16
            base = i * 16
            wvec = w_v[rr, pl.ds(cb, 16)]
            a0 = rows_v[base + 0] * wvec[0]
            a1 = rows_v[base + 1] * wvec[1]
            a2 = rows_v[base + 2] * wvec[2]
            a3 = rows_v[base + 3] * wvec[3]
            for l in range(4, 16, 4):
                a0 = a0 + rows_v[base + l] * wvec[l]
                a1 = a1 + rows_v[base + l + 1] * wvec[l + 1]
                a2 = a2 + rows_v[base + l + 2] * wvec[l + 2]
                a3 = a3 + rows_v[base + l + 3] * wvec[l + 3]
            out_v[i] = (a0 + a1) + (a2 + a3)
            return c2

        lax.fori_loop(0, KP, pair, 0)
        pltpu.sync_copy(out_v, out_ref.at[pl.ds(wid * PW + ch * KP, KP)])
        return carry

    lax.fori_loop(0, NCH, chunk, 0)


@functools.cache
def _sc_call():
    return functools.partial(
        pl.kernel,
        out_type=jax.ShapeDtypeStruct((NPAIR, HD), jnp.float32),
        mesh=plsc.VectorSubcoreMesh(core_axis_name="c", subcore_axis_name="s"),
        compiler_params=pltpu.CompilerParams(use_tc_tiling_on_sc=False),
        scratch_types=[
            pltpu.VMEM((NJ, 128), jnp.int32),
            pltpu.VMEM((NJ, 128), jnp.float32),
            pltpu.VMEM((NJ * 128, HD), jnp.float32),
            pltpu.VMEM((KP, HD), jnp.float32),
            pltpu.SemaphoreType.DMA,
        ],
    )(_sc_body)


def kernel(query, Wv, bv, Woff, boff, Wattn, battn, Wout, bout):
    q2 = query[0]
    wox = Woff[:, 0::2]
    woy = Woff[:, 1::2]
    box = boff[0::2][None, :]
    boy = boff[1::2][None, :]
    g4 = jnp.asarray(_G4)
    p0, p1, p2, p3 = (jnp.asarray(_P[c]) for c in range(4))

    value, widx, wwts = _pre_call(
        q2, Wv, bv[None, :], wox, box, woy, boy, Wattn, battn[None, :],
        g4, p0, p1, p2, p3)

    srows = _sc_call()(value.reshape(NPAIR, HD), widx, wwts)

    out = _post_call(srows.reshape(NQ, E), Wout, bout[None, :], q2)
    return out[None]


# speedup vs baseline: 2354.9310x; 1.8451x over previous
import functools

import numpy as np
import jax
import jax.numpy as jnp
from jax import lax
from jax.experimental import pallas as pl
from jax.experimental.pallas import tpu as pltpu
from jax.experimental.pallas import tpu_sc as plsc

BEV = 200
NQ = BEV * BEV
E = 128
NH = 8
HD = 16
NPAIR = NQ * NH

BLK = 1000
GRID = NQ // BLK

NC, NS = 2, 16
NW = NC * NS
PW = NPAIR // NW
KP = 200
NCH = PW // KP
NJ = KP * 16 // 128
TILE_IDXROWS = PW * 16 // 128

_PREC = jax.lax.Precision.DEFAULT

_LANE = np.arange(128)
_LH = _LANE // 16
_LC = (_LANE % 16) // 4
_LP = _LANE % 4
_SRCX = _LH * 8 + _LP * 2 + 0
_SRCY = _LH * 8 + _LP * 2 + 1
_SRCA = _LH * 4 + _LP
_G16 = np.kron(np.eye(NH, dtype=np.float32), np.ones((16, 16), np.float32))


def _dot(a, b):
    return jax.lax.dot_general(a, b, (((1,), (0,)), ((), ())),
                               precision=_PREC,
                               preferred_element_type=jnp.float32)


def _pre_body(q_ref, wv_ref, bv_ref, wox_ref, box_ref, woy_ref, boy_ref,
              wat_ref, bat_ref, g16_ref, val_ref, idx_ref, wts_ref):
    i = pl.program_id(0)
    q = q_ref[...]
    val_ref[...] = _dot(q, wv_ref[...]) + bv_ref[...]

    ox = _dot(q, wox_ref[...]) + box_ref[...]
    oy = _dot(q, woy_ref[...]) + boy_ref[...]
    lg = _dot(q, wat_ref[...]) + bat_ref[...]
    m = jnp.max(lg, axis=1, keepdims=True)
    e = jnp.exp(lg - m)
    aw = 4.0 * e / _dot(e, g16_ref[...])

    lane = lax.broadcasted_iota(jnp.int32, (BLK, E), 1)
    xsel = ((lane % 16) // 4 % 2).astype(jnp.float32)
    ysel = ((lane % 16) // 8).astype(jnp.float32)
    hf = (lane // 16).astype(jnp.float32)
    qi = i * BLK + lax.broadcasted_iota(jnp.int32, (BLK, E), 0)
    refx = (qi % BEV).astype(jnp.float32) * (1.0 / 199.0)
    refy = (qi // BEV).astype(jnp.float32) * (1.0 / 199.0)

    x = refx * 200.0 - 0.5 + ox
    y = refy * 200.0 - 0.5 + oy
    x0 = jnp.floor(x)
    y0 = jnp.floor(y)
    fx = x - x0
    fy = y - y0
    xf = x0 + xsel
    yf = y0 + ysel
    wx = (1.0 - xsel) + (2.0 * xsel - 1.0) * fx
    wy = (1.0 - ysel) + (2.0 * ysel - 1.0) * fy
    v = ((xf >= 0.0) & (xf <= 199.0) & (yf >= 0.0) & (yf <= 199.0))
    xc = jnp.clip(xf, 0.0, 199.0)
    yc = jnp.clip(yf, 0.0, 199.0)

    idx_ref[...] = ((yc * 200.0 + xc) * 8.0 + hf).astype(jnp.int32)
    wts_ref[...] = aw * wx * wy * v.astype(jnp.float32)


def _post_body(s_ref, wo_ref, bo_ref, q_ref, out_ref):
    out_ref[...] = (_dot(s_ref[...], wo_ref[...]) + bo_ref[...]
                    + 2.0 * q_ref[...])


def _full(shape):
    return pl.BlockSpec(shape, lambda i: (0, 0))


_pre_call = pl.pallas_call(
    _pre_body,
    grid=(GRID,),
    in_specs=[
        pl.BlockSpec((BLK, E), lambda i: (i, 0)),
        _full((E, E)), _full((1, E)),
        _full((E, E)), _full((1, E)),
        _full((E, E)), _full((1, E)),
        _full((E, E)), _full((1, E)),
        _full((E, E)),
    ],
    out_specs=[
        pl.BlockSpec((BLK, E), lambda i: (i, 0)),
        pl.BlockSpec((BLK, E), lambda i: (i, 0)),
        pl.BlockSpec((BLK, E), lambda i: (i, 0)),
    ],
    out_shape=[
        jax.ShapeDtypeStruct((NQ, E), jnp.float32),
        jax.ShapeDtypeStruct((NQ, E), jnp.int32),
        jax.ShapeDtypeStruct((NQ, E), jnp.float32),
    ],
)

_post_call = pl.pallas_call(
    _post_body,
    grid=(GRID,),
    in_specs=[
        pl.BlockSpec((BLK, E), lambda i: (i, 0)),
        _full((E, E)), _full((1, E)),
        pl.BlockSpec((BLK, E), lambda i: (i, 0)),
    ],
    out_specs=pl.BlockSpec((BLK, E), lambda i: (i, 0)),
    out_shape=jax.ShapeDtypeStruct((NQ, E), jnp.float32),
)


def _sc_body(tab_ref, idx_ref, w_ref, out_ref, idx_v, w_v, rows_v, out_v, sem):
    wid = lax.axis_index("s") * NC + lax.axis_index("c")

    def chunk(ch, carry):
        rowb = wid * TILE_IDXROWS + ch * NJ
        pltpu.sync_copy(idx_ref.at[pl.ds(rowb, NJ)], idx_v)
        pltpu.sync_copy(w_ref.at[pl.ds(rowb, NJ)], w_v)
        cps = [pltpu.async_copy(tab_ref.at[idx_v.at[j]],
                                rows_v.at[pl.ds(j * 128, 128)], sem)
               for j in range(NJ)]
        for cp in cps:
            cp.wait()

        def pair(i, c2):
            rr = i // 8
            cb = (i % 8) * 16
            base = i * 16
            wvec = w_v[rr, pl.ds(cb, 16)]
            a0 = rows_v[base + 0] * wvec[0]
            a1 = rows_v[base + 1] * wvec[1]
            a2 = rows_v[base + 2] * wvec[2]
            a3 = rows_v[base + 3] * wvec[3]
            for l in range(4, 16, 4):
                a0 = a0 + rows_v[base + l] * wvec[l]
                a1 = a1 + rows_v[base + l + 1] * wvec[l + 1]
                a2 = a2 + rows_v[base + l + 2] * wvec[l + 2]
                a3 = a3 + rows_v[base + l + 3] * wvec[l + 3]
            out_v[i] = (a0 + a1) + (a2 + a3)
            return c2

        lax.fori_loop(0, KP, pair, 0)
        pltpu.sync_copy(out_v, out_ref.at[pl.ds(wid * PW + ch * KP, KP)])
        return carry

    lax.fori_loop(0, NCH, chunk, 0)


@functools.cache
def _sc_call():
    return functools.partial(
        pl.kernel,
        out_type=jax.ShapeDtypeStruct((NPAIR, HD), jnp.float32),
        mesh=plsc.VectorSubcoreMesh(core_axis_name="c", subcore_axis_name="s"),
        compiler_params=pltpu.CompilerParams(use_tc_tiling_on_sc=False),
        scratch_types=[
            pltpu.VMEM((NJ, 128), jnp.int32),
            pltpu.VMEM((NJ, 128), jnp.float32),
            pltpu.VMEM((NJ * 128, HD), jnp.float32),
            pltpu.VMEM((KP, HD), jnp.float32),
            pltpu.SemaphoreType.DMA,
        ],
    )(_sc_body)


def kernel(query, Wv, bv, Woff, boff, Wattn, battn, Wout, bout):
    q2 = query[0]
    wox = Woff[:, _SRCX]
    woy = Woff[:, _SRCY]
    box = boff[_SRCX][None, :]
    boy = boff[_SRCY][None, :]
    wat = Wattn[:, _SRCA]
    bat = battn[_SRCA][None, :]
    g16 = jnp.asarray(_G16)

    value, widx, wwts = _pre_call(
        q2, Wv, bv[None, :], wox, box, woy, boy, wat, bat, g16)

    srows = _sc_call()(value.reshape(NPAIR, HD), widx, wwts)

    out = _post_call(srows.reshape(NQ, E), Wout, bout[None, :], q2)
    return out[None]


# speedup vs baseline: 4018.2069x; 1.7063x over previous
import functools

import numpy as np
import jax
import jax.numpy as jnp
from jax import lax
from jax.experimental import pallas as pl
from jax.experimental.pallas import tpu as pltpu
from jax.experimental.pallas import tpu_sc as plsc

BEV = 200
NQ = BEV * BEV
E = 128
NH = 8
HD = 16
NPAIR = NQ * NH

BLK = 1000
GRID = NQ // BLK

NC, NS = 2, 16
NW = NC * NS
PW = NPAIR // NW
KP = 200
NCH = PW // KP
NJ = KP * 16 // 128
TILE_IDXROWS = PW * 16 // 128

_PREC = jax.lax.Precision.DEFAULT

_LANE = np.arange(128)
_LH = _LANE // 16
_LC = (_LANE % 16) // 4
_LP = _LANE % 4
_SRCX = _LH * 8 + _LP * 2 + 0
_SRCY = _LH * 8 + _LP * 2 + 1
_SRCA = _LH * 4 + _LP
_G16 = np.kron(np.eye(NH, dtype=np.float32), np.ones((16, 16), np.float32))


def _dot(a, b):
    return jax.lax.dot_general(a, b, (((1,), (0,)), ((), ())),
                               precision=_PREC,
                               preferred_element_type=jnp.float32)


def _pre_body(q_ref, wv_ref, bv_ref, wox_ref, box_ref, woy_ref, boy_ref,
              wat_ref, bat_ref, g16_ref, val_ref, idx_ref, wts_ref):
    i = pl.program_id(0)
    q = q_ref[...]
    val_ref[...] = _dot(q, wv_ref[...]) + bv_ref[...]

    ox = _dot(q, wox_ref[...]) + box_ref[...]
    oy = _dot(q, woy_ref[...]) + boy_ref[...]
    lg = _dot(q, wat_ref[...]) + bat_ref[...]
    m = jnp.max(lg, axis=1, keepdims=True)
    e = jnp.exp(lg - m)
    aw = 4.0 * e / _dot(e, g16_ref[...])

    lane = lax.broadcasted_iota(jnp.int32, (BLK, E), 1)
    xsel = ((lane % 16) // 4 % 2).astype(jnp.float32)
    ysel = ((lane % 16) // 8).astype(jnp.float32)
    hf = (lane // 16).astype(jnp.float32)
    qi = i * BLK + lax.broadcasted_iota(jnp.int32, (BLK, E), 0)
    refx = (qi % BEV).astype(jnp.float32) * (1.0 / 199.0)
    refy = (qi // BEV).astype(jnp.float32) * (1.0 / 199.0)

    x = refx * 200.0 - 0.5 + ox
    y = refy * 200.0 - 0.5 + oy
    x0 = jnp.floor(x)
    y0 = jnp.floor(y)
    fx = x - x0
    fy = y - y0
    xf = x0 + xsel
    yf = y0 + ysel
    wx = (1.0 - xsel) + (2.0 * xsel - 1.0) * fx
    wy = (1.0 - ysel) + (2.0 * ysel - 1.0) * fy
    v = ((xf >= 0.0) & (xf <= 199.0) & (yf >= 0.0) & (yf <= 199.0))
    xc = jnp.clip(xf, 0.0, 199.0)
    yc = jnp.clip(yf, 0.0, 199.0)

    idx_ref[...] = ((yc * 200.0 + xc) * 8.0 + hf).astype(jnp.int32)
    wts_ref[...] = aw * wx * wy * v.astype(jnp.float32)


def _post_body(s_ref, wo_ref, bo_ref, q_ref, out_ref):
    out_ref[...] = (_dot(s_ref[...], wo_ref[...]) + bo_ref[...]
                    + 2.0 * q_ref[...])


def _full(shape):
    return pl.BlockSpec(shape, lambda i: (0, 0))


_pre_call = pl.pallas_call(
    _pre_body,
    grid=(GRID,),
    in_specs=[
        pl.BlockSpec((BLK, E), lambda i: (i, 0)),
        _full((E, E)), _full((1, E)),
        _full((E, E)), _full((1, E)),
        _full((E, E)), _full((1, E)),
        _full((E, E)), _full((1, E)),
        _full((E, E)),
    ],
    out_specs=[
        pl.BlockSpec((BLK, E), lambda i: (i, 0)),
        pl.BlockSpec((BLK, E), lambda i: (i, 0)),
        pl.BlockSpec((BLK, E), lambda i: (i, 0)),
    ],
    out_shape=[
        jax.ShapeDtypeStruct((NQ, E), jnp.float32),
        jax.ShapeDtypeStruct((NQ, E), jnp.int32),
        jax.ShapeDtypeStruct((NQ, E), jnp.float32),
    ],
)

_post_call = pl.pallas_call(
    _post_body,
    grid=(GRID,),
    in_specs=[
        pl.BlockSpec((BLK, E), lambda i: (i, 0)),
        _full((E, E)), _full((1, E)),
        pl.BlockSpec((BLK, E), lambda i: (i, 0)),
    ],
    out_specs=pl.BlockSpec((BLK, E), lambda i: (i, 0)),
    out_shape=jax.ShapeDtypeStruct((NQ, E), jnp.float32),
)


def _sc_body(tab_ref, idx_ref, w_ref, out_ref,
             idx_v, w_v, rows_v, out_v, sidx, sgat, sout):
    wid = lax.axis_index("s") * NC + lax.axis_index("c")
    pairbase = wid * PW
    idxbase = wid * TILE_IDXROWS

    def fire_idx(g):
        r = lax.rem(g, 3)
        pltpu.async_copy(idx_ref.at[pl.ds(idxbase + g * NJ, NJ)],
                         idx_v.at[pl.ds(r * NJ, NJ)], sidx)
        pltpu.async_copy(w_ref.at[pl.ds(pairbase + g * KP, KP)],
                         w_v.at[pl.ds(r * KP, KP)], sidx)

    def drain_idx():
        pltpu.make_async_copy(idx_ref.at[pl.ds(0, NJ)],
                              idx_v.at[pl.ds(0, NJ)], sidx).wait()
        pltpu.make_async_copy(w_ref.at[pl.ds(0, KP)],
                              w_v.at[pl.ds(0, KP)], sidx).wait()

    def fire_gat(g, b):
        r = lax.rem(g, 3)
        for j in range(NJ):
            pltpu.async_copy(tab_ref.at[idx_v.at[r * NJ + j]],
                             rows_v.at[pl.ds(b * KP * 16 + j * 128, 128)],
                             sgat)

    def drain_gat():
        pltpu.make_async_copy(tab_ref.at[pl.ds(0, KP * 16)],
                              rows_v.at[pl.ds(0, KP * 16)], sgat).wait()

    def drain_out():
        pltpu.make_async_copy(out_v.at[pl.ds(0, KP)],
                              out_ref.at[pl.ds(0, KP)], sout).wait()

    fire_idx(0)
    drain_idx()
    fire_gat(0, 0)
    fire_idx(1)

    def loop(g, carry):
        b = lax.rem(g, 2)

        @pl.when(g + 1 < NCH)
        def _():
            drain_idx()
            fire_gat(g + 1, 1 - b)

        @pl.when(g + 2 < NCH)
        def _():
            fire_idx(g + 2)

        @pl.when(g >= 2)
        def _():
            drain_out()

        drain_gat()

        rbase = b * KP * 16
        wbase = lax.rem(g, 3) * KP
        obase = b * KP

        def pair2(i2, c2):
            i = 2 * i2
            base = rbase + i * 16
            wv0 = w_v[wbase + i]
            wv1 = w_v[wbase + i + 1]
            a0 = rows_v[base + 0] * wv0[0]
            a1 = rows_v[base + 1] * wv0[1]
            a2 = rows_v[base + 2] * wv0[2]
            a3 = rows_v[base + 3] * wv0[3]
            b0 = rows_v[base + 16] * wv1[0]
            b1 = rows_v[base + 17] * wv1[1]
            b2 = rows_v[base + 18] * wv1[2]
            b3 = rows_v[base + 19] * wv1[3]
            for l in range(4, 16, 4):
                a0 = a0 + rows_v[base + l] * wv0[l]
                a1 = a1 + rows_v[base + l + 1] * wv0[l + 1]
                a2 = a2 + rows_v[base + l + 2] * wv0[l + 2]
                a3 = a3 + rows_v[base + l + 3] * wv0[l + 3]
                b0 = b0 + rows_v[base + 16 + l] * wv1[l]
                b1 = b1 + rows_v[base + 17 + l] * wv1[l + 1]
                b2 = b2 + rows_v[base + 18 + l] * wv1[l + 2]
                b3 = b3 + rows_v[base + 19 + l] * wv1[l + 3]
            out_v[obase + i] = (a0 + a1) + (a2 + a3)
            out_v[obase + i + 1] = (b0 + b1) + (b2 + b3)
            return c2

        lax.fori_loop(0, KP // 2, pair2, 0)
        pltpu.async_copy(out_v.at[pl.ds(obase, KP)],
                         out_ref.at[pl.ds(pairbase + g * KP, KP)], sout)
        return carry

    lax.fori_loop(0, NCH, loop, 0)
    drain_out()
    drain_out()


@functools.cache
def _sc_call():
    return functools.partial(
        pl.kernel,
        out_type=jax.ShapeDtypeStruct((NPAIR, HD), jnp.float32),
        mesh=plsc.VectorSubcoreMesh(core_axis_name="c", subcore_axis_name="s"),
        compiler_params=pltpu.CompilerParams(use_tc_tiling_on_sc=False),
        scratch_types=[
            pltpu.VMEM((3 * NJ, 128), jnp.int32),
            pltpu.VMEM((3 * KP, HD), jnp.float32),
            pltpu.VMEM((2 * KP * 16, HD), jnp.float32),
            pltpu.VMEM((2 * KP, HD), jnp.float32),
            pltpu.SemaphoreType.DMA,
            pltpu.SemaphoreType.DMA,
            pltpu.SemaphoreType.DMA,
        ],
    )(_sc_body)


def kernel(query, Wv, bv, Woff, boff, Wattn, battn, Wout, bout):
    q2 = query[0]
    wox = Woff[:, _SRCX]
    woy = Woff[:, _SRCY]
    box = boff[_SRCX][None, :]
    boy = boff[_SRCY][None, :]
    wat = Wattn[:, _SRCA]
    bat = battn[_SRCA][None, :]
    g16 = jnp.asarray(_G16)

    value, widx, wwts = _pre_call(
        q2, Wv, bv[None, :], wox, box, woy, boy, wat, bat, g16)

    srows = _sc_call()(value.reshape(NPAIR, HD), widx,
                       wwts.reshape(NPAIR, HD))

    out = _post_call(srows.reshape(NQ, E), Wout, bout[None, :], q2)
    return out[None]
